# baseline (speedup 1.0000x reference)
import functools

import jax
import jax.numpy as jnp
from jax import lax
from jax.experimental import pallas as pl
from jax.experimental.pallas import tpu as pltpu
from jax.experimental.pallas import tpu_sc as plsc

N = 10000
E = 320000
D_IN = 128
D_HID = 64
D_HALF = D_HID // 2

NC = 2
NS = 16
NPAD = 10240
ROWS_PER_TILE = NPAD // NS
CHUNK = 256
NCHUNKS = E // CHUNK
CPT = NCHUNKS // NS
XTRA = NCHUNKS - CPT * NS
KBUF = 4
RINGROUNDS = (CPT + 1 + KBUF - 1) // KBUF
BR = 1000

DCPT = (NCHUNKS // 2) // NS
DXTRA = NCHUNKS // 2 - DCPT * NS

_mesh = plsc.VectorSubcoreMesh(
    core_axis_name="c", subcore_axis_name="s", num_cores=NC, num_subcores=NS)
_sc_params = pltpu.CompilerParams(use_tc_tiling_on_sc=False)


@functools.partial(
    pl.kernel,
    out_type=jax.ShapeDtypeStruct((NC, NPAD), jnp.float32),
    mesh=_mesh,
    scratch_types=[
        pltpu.VMEM((DCPT + 1, CHUNK), jnp.int32),
        pltpu.VMEM((CHUNK,), jnp.float32),
        pltpu.VMEM((ROWS_PER_TILE,), jnp.float32),
        pltpu.VMEM_SHARED((NPAD,), jnp.float32),
    ],
    compiler_params=_sc_params,
)
def _deg_kernel(ei_hbm, out_hbm, dst_v, ones_v, zfill_v, acc_sh):
    cid = lax.axis_index("c")
    sid = lax.axis_index("s")
    c0 = cid * (NCHUNKS // 2) + sid * DCPT + jnp.minimum(sid, DXTRA)
    nch = DCPT + jnp.where(sid < DXTRA, 1, 0)
    pltpu.sync_copy(ei_hbm.at[1, pl.ds(c0, DCPT)], dst_v.at[pl.ds(0, DCPT)])

    @pl.when(sid < DXTRA)
    def _():
        pltpu.sync_copy(ei_hbm.at[1, pl.ds(c0 + DCPT, 1)],
                        dst_v.at[pl.ds(DCPT, 1)])

    def fill_ones(i, _):
        ones_v[pl.ds(i * 16, 16)] = jnp.ones((16,), jnp.float32)
        return 0
    lax.fori_loop(0, CHUNK // 16, fill_ones, 0)

    def fill_zero(i, _):
        zfill_v[pl.ds(i * 16, 16)] = jnp.zeros((16,), jnp.float32)
        return 0
    lax.fori_loop(0, ROWS_PER_TILE // 16, fill_zero, 0)
    row0 = sid * ROWS_PER_TILE
    pltpu.sync_copy(zfill_v, acc_sh.at[pl.ds(row0, ROWS_PER_TILE)])
    plsc.subcore_barrier()

    def scatter_ones(ch, _):
        @pl.when(ch < nch)
        def _():
            pltpu.sync_copy(ones_v, acc_sh.at[dst_v.at[ch]], add=True)
        return 0
    lax.fori_loop(0, DCPT + 1, scatter_ones, 0)
    plsc.subcore_barrier()
    pltpu.sync_copy(acc_sh.at[pl.ds(row0, ROWS_PER_TILE)],
                    out_hbm.at[cid, pl.ds(row0, ROWS_PER_TILE)])


_PIECES = []
_off = 0
while _off < ROWS_PER_TILE:
    _sz = min(CHUNK, ROWS_PER_TILE - _off)
    _PIECES.append((_off, _sz))
    _off += _sz


@functools.partial(
    pl.kernel,
    out_type=jax.ShapeDtypeStruct((NPAD, D_HID), jnp.float32),
    mesh=_mesh,
    scratch_types=[
        pltpu.VMEM((CPT + 1, CHUNK), jnp.int32),
        pltpu.VMEM((CPT + 1, CHUNK), jnp.int32),
        pltpu.VMEM((KBUF, CHUNK, D_HALF), jnp.float32),
        pltpu.VMEM((ROWS_PER_TILE, 16), jnp.float32),
        pltpu.VMEM((D_HALF,), jnp.float32),
        pltpu.VMEM_SHARED((NPAD, D_HALF), jnp.float32),
        pltpu.VMEM_SHARED((NPAD, D_HALF), jnp.float32),
        pltpu.SemaphoreType.DMA,
        pltpu.SemaphoreType.DMA,
    ],
    compiler_params=_sc_params,
)
def _gcn2x_kernel(z0_hbm, disx_hbm, ei_hbm, b1_hbm, q_hbm,
                  src_v, dst_v, bufs, disx_v, b1_v,
                  acc_sh, z_sh, sg, ss):
    cid = lax.axis_index("c")
    sid = lax.axis_index("s")
    col0 = cid * D_HALF
    c0 = sid * CPT + jnp.minimum(sid, XTRA)
    nch = CPT + jnp.where(sid < XTRA, 1, 0)
    row0 = sid * ROWS_PER_TILE
    pltpu.sync_copy(ei_hbm.at[0, pl.ds(c0, CPT)], src_v.at[pl.ds(0, CPT)])
    pltpu.sync_copy(ei_hbm.at[1, pl.ds(c0, CPT)], dst_v.at[pl.ds(0, CPT)])

    @pl.when(sid < XTRA)
    def _():
        pltpu.sync_copy(ei_hbm.at[0, pl.ds(c0 + CPT, 1)],
                        src_v.at[pl.ds(CPT, 1)])
        pltpu.sync_copy(ei_hbm.at[1, pl.ds(c0 + CPT, 1)],
                        dst_v.at[pl.ds(CPT, 1)])

    pltpu.sync_copy(disx_hbm.at[pl.ds(row0, ROWS_PER_TILE)], disx_v)
    pltpu.sync_copy(b1_hbm.at[pl.ds(col0, D_HALF)], b1_v)

    def scale_rows(first):
        for i, (off, sz) in enumerate(_PIECES):
            b = i % KBUF
            if first:
                pltpu.sync_copy(
                    z0_hbm.at[pl.ds(row0 + off, sz), pl.ds(col0, D_HALF)],
                    bufs.at[b, pl.ds(0, sz)])
            else:
                pltpu.sync_copy(acc_sh.at[pl.ds(row0 + off, sz)],
                                bufs.at[b, pl.ds(0, sz)])

            def row_body(r, _):
                d = disx_v[off + r, pl.ds(0, 16)]
                for cv in range(D_HALF // 16):
                    s = pl.ds(cv * 16, 16)
                    if first:
                        bufs[b, r, s] = bufs[b, r, s] * d
                    else:
                        v = bufs[b, r, s] * d + b1_v[pl.ds(cv * 16, 16)]
                        bufs[b, r, s] = jnp.maximum(v, 0.0) * d
                return 0
            lax.fori_loop(0, sz, row_body, 0)
            pltpu.sync_copy(bufs.at[b, pl.ds(0, sz)],
                            z_sh.at[pl.ds(row0 + off, sz)])
            pltpu.sync_copy(bufs.at[b, pl.ds(0, sz)],
                            acc_sh.at[pl.ds(row0 + off, sz)])

    scale_rows(True)
    plsc.subcore_barrier()

    def edge_pass():
        for j in range(KBUF):
            pltpu.async_copy(z_sh.at[src_v.at[j]], bufs.at[j], sg)

        def round_body(p, _):
            base = p * KBUF
            for j in range(KBUF):
                ch = base + j

                @pl.when(ch < nch)
                def _():
                    pltpu.make_async_copy(z_sh.at[src_v.at[ch]],
                                          bufs.at[j], sg).wait()
                    pltpu.async_copy(bufs.at[j], acc_sh.at[dst_v.at[ch]],
                                     ss, add=True)
            for j in range(KBUF):
                ch = base + j
                ch2 = ch + KBUF

                @pl.when(ch < nch)
                def _():
                    pltpu.make_async_copy(bufs.at[j],
                                          acc_sh.at[dst_v.at[ch]],
                                          ss).wait()

                @pl.when(ch2 < nch)
                def _():
                    pltpu.async_copy(z_sh.at[src_v.at[ch2]],
                                     bufs.at[j], sg)
            return 0
        lax.fori_loop(0, RINGROUNDS, round_body, 0)

    edge_pass()
    plsc.subcore_barrier()
    scale_rows(False)
    plsc.subcore_barrier()
    edge_pass()
    plsc.subcore_barrier()

    for i, (off, sz) in enumerate(_PIECES):
        b = i % KBUF
        pltpu.sync_copy(acc_sh.at[pl.ds(row0 + off, sz)],
                        bufs.at[b, pl.ds(0, sz)])
        pltpu.sync_copy(bufs.at[b, pl.ds(0, sz)],
                        q_hbm.at[pl.ds(row0 + off, sz), pl.ds(col0, D_HALF)])


def _mm_body(x_ref, w_ref, h_ref):
    h_ref[...] = jnp.dot(x_ref[...], w_ref[...],
                         preferred_element_type=jnp.float32)


def _disk_body(d0_ref, d1_ref, dis_ref, disx_ref):
    dis = lax.rsqrt(d0_ref[...] + d1_ref[...] + 1.0)
    dis_ref[...] = dis
    disx_ref[...] = jnp.broadcast_to(dis, (dis.shape[0], 16))


def _final_body(q_ref, dis_ref, wm_ref, bm_ref, wl_ref, bl_ref,
                mu_ref, ls_ref):
    g = q_ref[...] * dis_ref[...]
    mu_ref[...] = jnp.dot(g, wm_ref[...],
                          preferred_element_type=jnp.float32) + bm_ref[...]
    ls_ref[...] = jnp.dot(g, wl_ref[...],
                          preferred_element_type=jnp.float32) + bl_ref[...]


def _row_spec(d):
    return pl.BlockSpec((BR, d), lambda i: (i, 0))


def _full_spec(r, c):
    return pl.BlockSpec((r, c), lambda i: (0, 0))


@jax.jit
def kernel(x, edge_index, W1, b1, W_mu, b_mu, W_ls, b_ls):
    ei = edge_index.astype(jnp.int32).reshape(2, NCHUNKS, CHUNK)

    degp = _deg_kernel(ei)
    d0 = degp[0][:, None]
    d1 = degp[1][:, None]

    grid = (N // BR,)
    h0 = pl.pallas_call(
        _mm_body,
        grid=grid,
        in_specs=[_row_spec(D_IN), _full_spec(D_IN, D_HID)],
        out_specs=_row_spec(D_HID),
        out_shape=jax.ShapeDtypeStruct((NPAD, D_HID), jnp.float32),
    )(x, W1)

    dis, disx = pl.pallas_call(
        _disk_body,
        in_specs=[pl.BlockSpec((NPAD, 1), lambda: (0, 0)),
                  pl.BlockSpec((NPAD, 1), lambda: (0, 0))],
        out_specs=[pl.BlockSpec((NPAD, 1), lambda: (0, 0)),
                   pl.BlockSpec((NPAD, 16), lambda: (0, 0))],
        out_shape=[jax.ShapeDtypeStruct((NPAD, 1), jnp.float32),
                   jax.ShapeDtypeStruct((NPAD, 16), jnp.float32)],
    )(d0, d1)

    q = _gcn2x_kernel(h0, disx, ei, b1)

    mu, ls = pl.pallas_call(
        _final_body,
        grid=grid,
        in_specs=[_row_spec(D_HID), _row_spec(1), _full_spec(D_HID, D_HID),
                  _full_spec(1, D_HID), _full_spec(D_HID, D_HID),
                  _full_spec(1, D_HID)],
        out_specs=[_row_spec(D_HID), _row_spec(D_HID)],
        out_shape=[jax.ShapeDtypeStruct((N, D_HID), jnp.float32),
                   jax.ShapeDtypeStruct((N, D_HID), jnp.float32)],
    )(q, dis, W_mu, b_mu[None, :], W_ls, b_ls[None, :])

    return mu, ls


# speedup vs baseline: 1.0015x; 1.0015x over previous
import functools

import jax
import jax.numpy as jnp
from jax import lax
from jax.experimental import pallas as pl
from jax.experimental.pallas import tpu as pltpu
from jax.experimental.pallas import tpu_sc as plsc

N = 10000
E = 320000
D_IN = 128
D_HID = 64
D_HALF = D_HID // 2

NC = 2
NS = 16
NPAD = 10240
ROWS_PER_TILE = NPAD // NS
CHUNK = 256
NCHUNKS = E // CHUNK
CPT = NCHUNKS // NS
XTRA = NCHUNKS - CPT * NS
KBUF = 4
RINGROUNDS = (CPT + 1 + KBUF - 1) // KBUF
BR = 1000

DCPT = (NCHUNKS // 2) // NS
DXTRA = NCHUNKS // 2 - DCPT * NS

_mesh = plsc.VectorSubcoreMesh(
    core_axis_name="c", subcore_axis_name="s", num_cores=NC, num_subcores=NS)
_sc_params = pltpu.CompilerParams(use_tc_tiling_on_sc=False)


@functools.partial(
    pl.kernel,
    out_type=jax.ShapeDtypeStruct((NC, NPAD), jnp.float32),
    mesh=_mesh,
    scratch_types=[
        pltpu.VMEM((DCPT + 1, CHUNK), jnp.int32),
        pltpu.VMEM((CHUNK,), jnp.float32),
        pltpu.VMEM((ROWS_PER_TILE,), jnp.float32),
        pltpu.VMEM_SHARED((NPAD,), jnp.float32),
    ],
    compiler_params=_sc_params,
)
def _deg_kernel(ei_hbm, out_hbm, dst_v, ones_v, zfill_v, acc_sh):
    cid = lax.axis_index("c")
    sid = lax.axis_index("s")
    c0 = cid * (NCHUNKS // 2) + sid * DCPT + jnp.minimum(sid, DXTRA)
    nch = DCPT + jnp.where(sid < DXTRA, 1, 0)
    pltpu.sync_copy(ei_hbm.at[1, pl.ds(c0, DCPT)], dst_v.at[pl.ds(0, DCPT)])

    @pl.when(sid < DXTRA)
    def _():
        pltpu.sync_copy(ei_hbm.at[1, pl.ds(c0 + DCPT, 1)],
                        dst_v.at[pl.ds(DCPT, 1)])

    def fill_ones(i, _):
        ones_v[pl.ds(i * 16, 16)] = jnp.ones((16,), jnp.float32)
        return 0
    lax.fori_loop(0, CHUNK // 16, fill_ones, 0)

    def fill_zero(i, _):
        zfill_v[pl.ds(i * 16, 16)] = jnp.zeros((16,), jnp.float32)
        return 0
    lax.fori_loop(0, ROWS_PER_TILE // 16, fill_zero, 0)
    row0 = sid * ROWS_PER_TILE
    pltpu.sync_copy(zfill_v, acc_sh.at[pl.ds(row0, ROWS_PER_TILE)])
    plsc.subcore_barrier()

    def scatter_ones(ch, _):
        @pl.when(ch < nch)
        def _():
            pltpu.sync_copy(ones_v, acc_sh.at[dst_v.at[ch]], add=True)
        return 0
    lax.fori_loop(0, DCPT + 1, scatter_ones, 0)
    plsc.subcore_barrier()
    pltpu.sync_copy(acc_sh.at[pl.ds(row0, ROWS_PER_TILE)],
                    out_hbm.at[cid, pl.ds(row0, ROWS_PER_TILE)])


_PIECES = []
_off = 0
while _off < ROWS_PER_TILE:
    _sz = min(CHUNK, ROWS_PER_TILE - _off)
    _PIECES.append((_off, _sz))
    _off += _sz


@functools.partial(
    pl.kernel,
    out_type=jax.ShapeDtypeStruct((NPAD, D_HID), jnp.float32),
    mesh=_mesh,
    scratch_types=[
        pltpu.VMEM((CPT + 1, CHUNK), jnp.int32),
        pltpu.VMEM((CPT + 1, CHUNK), jnp.int32),
        pltpu.VMEM((KBUF, CHUNK, D_HALF), jnp.float32),
        pltpu.VMEM((ROWS_PER_TILE, 16), jnp.float32),
        pltpu.VMEM((D_HALF,), jnp.float32),
        pltpu.VMEM_SHARED((NPAD, D_HALF), jnp.float32),
        pltpu.VMEM_SHARED((NPAD, D_HALF), jnp.float32),
        pltpu.SemaphoreType.DMA,
        pltpu.SemaphoreType.DMA,
    ],
    compiler_params=_sc_params,
)
def _gcn2x_kernel(h0_hbm, disx_hbm, ei_hbm, b1_hbm, q_hbm,
                  src_v, dst_v, bufs, disx_v, b1_v,
                  acc_sh, z_sh, sg, ss):
    cid = lax.axis_index("c")
    sid = lax.axis_index("s")
    col0 = cid * D_HALF
    c0 = sid * CPT + jnp.minimum(sid, XTRA)
    nch = CPT + jnp.where(sid < XTRA, 1, 0)
    row0 = sid * ROWS_PER_TILE
    pltpu.sync_copy(ei_hbm.at[0, pl.ds(c0, CPT)], src_v.at[pl.ds(0, CPT)])
    pltpu.sync_copy(ei_hbm.at[1, pl.ds(c0, CPT)], dst_v.at[pl.ds(0, CPT)])

    @pl.when(sid < XTRA)
    def _():
        pltpu.sync_copy(ei_hbm.at[0, pl.ds(c0 + CPT, 1)],
                        src_v.at[pl.ds(CPT, 1)])
        pltpu.sync_copy(ei_hbm.at[1, pl.ds(c0 + CPT, 1)],
                        dst_v.at[pl.ds(CPT, 1)])

    pltpu.sync_copy(disx_hbm.at[pl.ds(row0, ROWS_PER_TILE)], disx_v)
    pltpu.sync_copy(b1_hbm.at[pl.ds(col0, D_HALF)], b1_v)

    def scale_rows(first):
        for i, (off, sz) in enumerate(_PIECES):
            b = i % KBUF
            if first:
                pltpu.sync_copy(
                    h0_hbm.at[pl.ds(row0 + off, sz), pl.ds(col0, D_HALF)],
                    bufs.at[b, pl.ds(0, sz)])
            else:
                pltpu.sync_copy(acc_sh.at[pl.ds(row0 + off, sz)],
                                bufs.at[b, pl.ds(0, sz)])

            def row_body(r, _):
                d = disx_v[off + r, pl.ds(0, 16)]
                for cv in range(D_HALF // 16):
                    s = pl.ds(cv * 16, 16)
                    if first:
                        bufs[b, r, s] = bufs[b, r, s] * d
                    else:
                        v = bufs[b, r, s] * d + b1_v[pl.ds(cv * 16, 16)]
                        bufs[b, r, s] = jnp.maximum(v, 0.0) * d
                return 0
            lax.fori_loop(0, sz, row_body, 0)
            pltpu.sync_copy(bufs.at[b, pl.ds(0, sz)],
                            z_sh.at[pl.ds(row0 + off, sz)])
            pltpu.sync_copy(bufs.at[b, pl.ds(0, sz)],
                            acc_sh.at[pl.ds(row0 + off, sz)])

    scale_rows(True)
    plsc.subcore_barrier()

    def edge_pass():
        for j in range(KBUF):
            pltpu.async_copy(z_sh.at[src_v.at[j]], bufs.at[j], sg)

        def round_body(p, _):
            base = p * KBUF
            for j in range(KBUF):
                ch = base + j

                @pl.when(ch < nch)
                def _():
                    pltpu.make_async_copy(z_sh.at[src_v.at[ch]],
                                          bufs.at[j], sg).wait()
                    pltpu.async_copy(bufs.at[j], acc_sh.at[dst_v.at[ch]],
                                     ss, add=True)
            for j in range(KBUF):
                ch = base + j
                ch2 = ch + KBUF

                @pl.when(ch < nch)
                def _():
                    pltpu.make_async_copy(bufs.at[j],
                                          acc_sh.at[dst_v.at[ch]],
                                          ss).wait()

                @pl.when(ch2 < nch)
                def _():
                    pltpu.async_copy(z_sh.at[src_v.at[ch2]],
                                     bufs.at[j], sg)
            return 0
        lax.fori_loop(0, RINGROUNDS, round_body, 0)

    edge_pass()
    plsc.subcore_barrier()
    scale_rows(False)
    plsc.subcore_barrier()
    edge_pass()
    plsc.subcore_barrier()

    for i, (off, sz) in enumerate(_PIECES):
        b = i % KBUF
        pltpu.sync_copy(acc_sh.at[pl.ds(row0 + off, sz)],
                        bufs.at[b, pl.ds(0, sz)])
        pltpu.sync_copy(bufs.at[b, pl.ds(0, sz)],
                        q_hbm.at[pl.ds(row0 + off, sz), pl.ds(col0, D_HALF)])


def _mm_body(x_ref, w_ref, h_ref):
    h_ref[...] = jnp.dot(x_ref[...], w_ref[...],
                         preferred_element_type=jnp.float32)


def _disk_body(d0_ref, d1_ref, dis_ref, disx_ref):
    dis = lax.rsqrt(d0_ref[...] + d1_ref[...] + 1.0)
    dis_ref[...] = dis
    disx_ref[...] = jnp.broadcast_to(dis, (dis.shape[0], 16))


def _final_body(q_ref, dis_ref, wm_ref, bm_ref, wl_ref, bl_ref,
                mu_ref, ls_ref):
    g = q_ref[...] * dis_ref[...]
    mu_ref[...] = jnp.dot(g, wm_ref[...],
                          preferred_element_type=jnp.float32) + bm_ref[...]
    ls_ref[...] = jnp.dot(g, wl_ref[...],
                          preferred_element_type=jnp.float32) + bl_ref[...]


def _row_spec(d):
    return pl.BlockSpec((BR, d), lambda i: (i, 0))


def _full_spec(r, c):
    return pl.BlockSpec((r, c), lambda i: (0, 0))


@jax.jit
def kernel(x, edge_index, W1, b1, W_mu, b_mu, W_ls, b_ls):
    ei = edge_index.astype(jnp.int32).reshape(2, NCHUNKS, CHUNK)

    degp = _deg_kernel(ei)
    d0 = degp[0][:, None]
    d1 = degp[1][:, None]

    grid = (N // BR,)
    h0 = pl.pallas_call(
        _mm_body,
        grid=grid,
        in_specs=[_row_spec(D_IN), _full_spec(D_IN, D_HID)],
        out_specs=_row_spec(D_HID),
        out_shape=jax.ShapeDtypeStruct((NPAD, D_HID), jnp.float32),
    )(x, W1)

    dis, disx = pl.pallas_call(
        _disk_body,
        in_specs=[pl.BlockSpec((NPAD, 1), lambda: (0, 0)),
                  pl.BlockSpec((NPAD, 1), lambda: (0, 0))],
        out_specs=[pl.BlockSpec((NPAD, 1), lambda: (0, 0)),
                   pl.BlockSpec((NPAD, 16), lambda: (0, 0))],
        out_shape=[jax.ShapeDtypeStruct((NPAD, 1), jnp.float32),
                   jax.ShapeDtypeStruct((NPAD, 16), jnp.float32)],
    )(d0, d1)

    q = _gcn2x_kernel(h0, disx, ei, b1)

    mu, ls = pl.pallas_call(
        _final_body,
        grid=grid,
        in_specs=[_row_spec(D_HID), _row_spec(1), _full_spec(D_HID, D_HID),
                  _full_spec(1, D_HID), _full_spec(D_HID, D_HID),
                  _full_spec(1, D_HID)],
        out_specs=[_row_spec(D_HID), _row_spec(D_HID)],
        out_shape=[jax.ShapeDtypeStruct((N, D_HID), jnp.float32),
                   jax.ShapeDtypeStruct((N, D_HID), jnp.float32)],
    )(q, dis, W_mu, b_mu[None, :], W_ls, b_ls[None, :])

    return mu, ls
